# Initial kernel scaffold; baseline (speedup 1.0000x reference)
import jax
import jax.numpy as jnp
from jax.experimental import pallas as pl


def kernel(x1, x2, edge_index1, edge_index2, W1, b1, W2, b2):
    raise NotImplementedError("write your pallas kernel here")


# speedup vs baseline: 16.5408x; 16.5408x over previous
import functools

import jax
import jax.numpy as jnp
from jax import lax
from jax.experimental import pallas as pl
from jax.experimental.pallas import tpu as pltpu
from jax.experimental.pallas import tpu_sc as plsc

N = 10000
E = 320000
D = 128
NT = 16
CH = 128
NCH = 160
GRP = 16
NGRP = NCH // GRP
EPT = NCH * CH
EPAD = NT * EPT
NPAD = 10016
CB = 632
CBL = N - 15 * CB
CBLZ = NPAD - 15 * CB


@functools.cache
def _get_deg_kernel():
    mesh = plsc.VectorSubcoreMesh(core_axis_name="c", subcore_axis_name="s")
    return functools.partial(
        pl.kernel,
        mesh=mesh,
        out_type=jax.ShapeDtypeStruct((2, N, D), jnp.float32),
        scratch_types=[
            pltpu.VMEM((GRP, CH), jnp.int32),
            pltpu.VMEM((CH, D), jnp.float32),
            pltpu.VMEM_SHARED((NPAD, D), jnp.float32),
        ],
    )(_deg_body)


def _deg_body(dst_hbm, ones_hbm, out_hbm, dst_v, rows_v, acc_sh):
    c = lax.axis_index("c")
    s = lax.axis_index("s")
    pltpu.sync_copy(ones_hbm.at[pl.ds(0, CH)], rows_v)

    @pl.when(s < 15)
    def _():
        pltpu.sync_copy(ones_hbm.at[pl.ds(s * CB, CB)], acc_sh.at[pl.ds(s * CB, CB)])

    @pl.when(s == 15)
    def _():
        pltpu.sync_copy(ones_hbm.at[pl.ds(15 * CB, CBL)], acc_sh.at[pl.ds(15 * CB, CBL)])

    plsc.subcore_barrier()

    def grp_body(g, carry):
        pltpu.sync_copy(dst_hbm.at[c, s, pl.ds(g * GRP, GRP)], dst_v)

        def body(k, carry2):
            pltpu.sync_copy(rows_v, acc_sh.at[dst_v.at[k]], add=True)
            return carry2

        lax.fori_loop(0, GRP, body, 0)
        return carry

    lax.fori_loop(0, NGRP, grp_body, 0)
    plsc.subcore_barrier()

    @pl.when(s < 15)
    def _():
        pltpu.sync_copy(acc_sh.at[pl.ds(s * CB, CB)], out_hbm.at[c, pl.ds(s * CB, CB)])

    @pl.when(s == 15)
    def _():
        pltpu.sync_copy(acc_sh.at[pl.ds(15 * CB, CBL)], out_hbm.at[c, pl.ds(15 * CB, CBL)])


@functools.cache
def _get_agg_kernel():
    mesh = plsc.VectorSubcoreMesh(core_axis_name="c", subcore_axis_name="s")
    return functools.partial(
        pl.kernel,
        mesh=mesh,
        out_type=jax.ShapeDtypeStruct((2, N, D), jnp.float32),
        scratch_types=[
            pltpu.VMEM((GRP, CH), jnp.int32),
            pltpu.VMEM((GRP, CH), jnp.int32),
            pltpu.VMEM((CH, D), jnp.float32),
            pltpu.VMEM_SHARED((NPAD, D), jnp.float32),
            pltpu.SemaphoreType.DMA,
        ],
    )(_agg_body)


def _agg_body(hp_hbm, src_hbm, dst_hbm, out_hbm, src_v, dst_v, rows_v, acc_sh, sem):
    c = lax.axis_index("c")
    s = lax.axis_index("s")
    hp_c = hp_hbm.at[c]

    @pl.when(s < 15)
    def _():
        pltpu.sync_copy(hp_c.at[pl.ds(s * CB, CB)], acc_sh.at[pl.ds(s * CB, CB)])

    @pl.when(s == 15)
    def _():
        pltpu.sync_copy(hp_c.at[pl.ds(15 * CB, CBL)], acc_sh.at[pl.ds(15 * CB, CBL)])

    plsc.subcore_barrier()

    def grp_body(g, carry):
        pltpu.sync_copy(src_hbm.at[c, s, pl.ds(g * GRP, GRP)], src_v)
        pltpu.sync_copy(dst_hbm.at[c, s, pl.ds(g * GRP, GRP)], dst_v)

        def body(k, carry2):
            pltpu.async_copy(hp_c.at[src_v.at[k]], rows_v, sem).wait()
            pltpu.sync_copy(rows_v, acc_sh.at[dst_v.at[k]], add=True)
            return carry2

        lax.fori_loop(0, GRP, body, 0)
        return carry

    lax.fori_loop(0, NGRP, grp_body, 0)
    plsc.subcore_barrier()

    @pl.when(s < 15)
    def _():
        pltpu.sync_copy(acc_sh.at[pl.ds(s * CB, CB)], out_hbm.at[c, pl.ds(s * CB, CB)])

    @pl.when(s == 15)
    def _():
        pltpu.sync_copy(acc_sh.at[pl.ds(15 * CB, CBL)], out_hbm.at[c, pl.ds(15 * CB, CBL)])


_BM = 1000
_NB = N // _BM


def _tc1_body(x_ref, w_ref, deg_ref, hp_ref):
    dv = lax.rsqrt(deg_ref[0])
    h = jnp.dot(x_ref[0], w_ref[...], preferred_element_type=jnp.float32)
    hp_ref[0] = h * dv


def _tc1(x, w1, deg):
    return pl.pallas_call(
        _tc1_body,
        grid=(2, _NB),
        in_specs=[
            pl.BlockSpec((1, _BM, D), lambda g, i: (g, i, 0)),
            pl.BlockSpec((D, D), lambda g, i: (0, 0)),
            pl.BlockSpec((1, _BM, D), lambda g, i: (g, i, 0)),
        ],
        out_specs=pl.BlockSpec((1, _BM, D), lambda g, i: (g, i, 0)),
        out_shape=jax.ShapeDtypeStruct((2, N, D), jnp.float32),
    )(x, w1, deg)


def _tc2_body(agg_ref, deg_ref, b_ref, w_ref, out_ref):
    dv = lax.rsqrt(deg_ref[0])
    y = jnp.maximum(agg_ref[0] * dv + b_ref[...], 0.0)
    out_ref[0] = jnp.dot(y, w_ref[...], preferred_element_type=jnp.float32) * dv


def _tc2(agg, deg, b1, w2):
    return pl.pallas_call(
        _tc2_body,
        grid=(2, _NB),
        in_specs=[
            pl.BlockSpec((1, _BM, D), lambda g, i: (g, i, 0)),
            pl.BlockSpec((1, _BM, D), lambda g, i: (g, i, 0)),
            pl.BlockSpec((1, D), lambda g, i: (0, 0)),
            pl.BlockSpec((D, D), lambda g, i: (0, 0)),
        ],
        out_specs=pl.BlockSpec((1, _BM, D), lambda g, i: (g, i, 0)),
        out_shape=jax.ShapeDtypeStruct((2, N, D), jnp.float32),
    )(agg, deg, b1, w2)


def _tc3_body(agg_ref, deg_ref, b_ref, out_ref, sum_ref, sq_ref):
    p = pl.program_id(1)
    i = pl.program_id(2)
    dv = lax.rsqrt(deg_ref[0])
    h = agg_ref[0] * dv + b_ref[...]

    @pl.when((p == 0) & (i == 0))
    def _():
        sum_ref[...] = jnp.zeros_like(sum_ref)
        sq_ref[...] = jnp.zeros_like(sq_ref)

    @pl.when(p == 0)
    def _():
        sum_ref[...] += jnp.sum(h, axis=0, keepdims=True)
        sq_ref[...] += jnp.sum(h * h, axis=0, keepdims=True)
        out_ref[0] = h

    @pl.when(p == 1)
    def _():
        mean = sum_ref[...] / N
        var = (sq_ref[...] - N * mean * mean) / (N - 1)
        out_ref[0] = (h - mean) * lax.rsqrt(var)


def _tc3(agg, deg, b2):
    return pl.pallas_call(
        _tc3_body,
        grid=(2, 2, _NB),
        in_specs=[
            pl.BlockSpec((1, _BM, D), lambda g, p, i: (g, i, 0)),
            pl.BlockSpec((1, _BM, D), lambda g, p, i: (g, i, 0)),
            pl.BlockSpec((1, D), lambda g, p, i: (0, 0)),
        ],
        out_specs=pl.BlockSpec((1, _BM, D), lambda g, p, i: (g, i, 0)),
        out_shape=jax.ShapeDtypeStruct((2, N, D), jnp.float32),
        scratch_shapes=[
            pltpu.VMEM((1, D), jnp.float32),
            pltpu.VMEM((1, D), jnp.float32),
        ],
    )(agg, deg, b2)


def kernel(x1, x2, edge_index1, edge_index2, W1, b1, W2, b2):
    src = jnp.stack([edge_index1[0], edge_index2[0]])
    dst = jnp.stack([edge_index1[1], edge_index2[1]])
    extra = EPAD - E
    pad_src = (jnp.arange(extra, dtype=jnp.int32) * 97) % N
    pad_dst = N + (jnp.arange(extra, dtype=jnp.int32) % (NPAD - N))
    srcp = jnp.concatenate(
        [src, jnp.broadcast_to(pad_src, (2, extra))], axis=1
    ).reshape(2, NT, NCH, CH)
    dstp = jnp.concatenate(
        [dst, jnp.broadcast_to(pad_dst, (2, extra))], axis=1
    ).reshape(2, NT, NCH, CH)

    ones = jnp.ones((N, D), jnp.float32)
    x = jnp.stack([x1, x2])

    deg = _get_deg_kernel()(dstp, ones)
    hp1 = _tc1(x, W1, deg)
    agg = _get_agg_kernel()
    agg1 = agg(hp1, srcp, dstp)
    hp2 = _tc2(agg1, deg, b1.reshape(1, D), W2)
    agg2 = agg(hp2, srcp, dstp)
    z = _tc3(agg2, deg, b2.reshape(1, D))
    return z[0], z[1]


# speedup vs baseline: 22.4797x; 1.3590x over previous
import functools

import jax
import jax.numpy as jnp
from jax import lax
from jax.experimental import pallas as pl
from jax.experimental.pallas import tpu as pltpu
from jax.experimental.pallas import tpu_sc as plsc

N = 10000
E = 320000
D = 128
NT = 16
CH = 128
NCH = 160
GRP = 16
NGRP = NCH // GRP
EPT = NCH * CH
EPAD = NT * EPT
NPAD = 10016
CB = 632
CBL = N - 15 * CB
CBLZ = NPAD - 15 * CB


@functools.cache
def _get_deg_kernel():
    mesh = plsc.VectorSubcoreMesh(core_axis_name="c", subcore_axis_name="s")
    return functools.partial(
        pl.kernel,
        mesh=mesh,
        out_type=jax.ShapeDtypeStruct((2, N, D), jnp.float32),
        scratch_types=[
            pltpu.VMEM((GRP, CH), jnp.int32),
            pltpu.VMEM((CH, D), jnp.float32),
            pltpu.VMEM_SHARED((NPAD, D), jnp.float32),
        ],
    )(_deg_body)


def _deg_body(dst_hbm, ones_hbm, out_hbm, dst_v, rows_v, acc_sh):
    c = lax.axis_index("c")
    s = lax.axis_index("s")
    pltpu.sync_copy(ones_hbm.at[pl.ds(0, CH)], rows_v)

    @pl.when(s < 15)
    def _():
        pltpu.sync_copy(ones_hbm.at[pl.ds(s * CB, CB)], acc_sh.at[pl.ds(s * CB, CB)])

    @pl.when(s == 15)
    def _():
        pltpu.sync_copy(ones_hbm.at[pl.ds(15 * CB, CBL)], acc_sh.at[pl.ds(15 * CB, CBL)])

    plsc.subcore_barrier()

    def grp_body(g, carry):
        pltpu.sync_copy(dst_hbm.at[c, s, pl.ds(g * GRP, GRP)], dst_v)

        def body(k, carry2):
            pltpu.sync_copy(rows_v, acc_sh.at[dst_v.at[k]], add=True)
            return carry2

        lax.fori_loop(0, GRP, body, 0)
        return carry

    lax.fori_loop(0, NGRP, grp_body, 0)
    plsc.subcore_barrier()

    @pl.when(s < 15)
    def _():
        pltpu.sync_copy(acc_sh.at[pl.ds(s * CB, CB)], out_hbm.at[c, pl.ds(s * CB, CB)])

    @pl.when(s == 15)
    def _():
        pltpu.sync_copy(acc_sh.at[pl.ds(15 * CB, CBL)], out_hbm.at[c, pl.ds(15 * CB, CBL)])


@functools.cache
def _get_agg_kernel():
    mesh = plsc.VectorSubcoreMesh(core_axis_name="c", subcore_axis_name="s")
    return functools.partial(
        pl.kernel,
        mesh=mesh,
        out_type=jax.ShapeDtypeStruct((2, N, D), jnp.float32),
        scratch_types=[
            pltpu.VMEM((GRP, CH), jnp.int32),
            pltpu.VMEM((GRP, CH), jnp.int32),
            pltpu.VMEM((CH, D), jnp.float32),
            pltpu.VMEM((CH, D), jnp.float32),
            pltpu.VMEM_SHARED((NPAD, D), jnp.float32),
            pltpu.SemaphoreType.DMA,
            pltpu.SemaphoreType.DMA,
        ],
    )(_agg_body)


def _agg_body(hp_hbm, src_hbm, dst_hbm, out_hbm, src_v, dst_v, rows_a, rows_b,
              acc_sh, sem_a, sem_b):
    c = lax.axis_index("c")
    s = lax.axis_index("s")
    hp_c = hp_hbm.at[c]

    @pl.when(s < 15)
    def _():
        pltpu.sync_copy(hp_c.at[pl.ds(s * CB, CB)], acc_sh.at[pl.ds(s * CB, CB)])

    @pl.when(s == 15)
    def _():
        pltpu.sync_copy(hp_c.at[pl.ds(15 * CB, CBL)], acc_sh.at[pl.ds(15 * CB, CBL)])

    plsc.subcore_barrier()

    def grp_body(g, carry):
        pltpu.sync_copy(src_hbm.at[c, s, pl.ds(g * GRP, GRP)], src_v)
        pltpu.sync_copy(dst_hbm.at[c, s, pl.ds(g * GRP, GRP)], dst_v)
        pltpu.async_copy(hp_c.at[src_v.at[0]], rows_a, sem_a)

        def pair(kk, carry2):
            k0 = kk * 2
            pltpu.async_copy(hp_c.at[src_v.at[k0 + 1]], rows_b, sem_b)
            pltpu.make_async_copy(hp_c.at[src_v.at[k0]], rows_a, sem_a).wait()
            pltpu.sync_copy(rows_a, acc_sh.at[dst_v.at[k0]], add=True)

            @pl.when(kk < GRP // 2 - 1)
            def _():
                pltpu.async_copy(hp_c.at[src_v.at[k0 + 2]], rows_a, sem_a)

            pltpu.make_async_copy(hp_c.at[src_v.at[k0 + 1]], rows_b, sem_b).wait()
            pltpu.sync_copy(rows_b, acc_sh.at[dst_v.at[k0 + 1]], add=True)
            return carry2

        lax.fori_loop(0, GRP // 2, pair, 0)
        return carry

    lax.fori_loop(0, NGRP, grp_body, 0)
    plsc.subcore_barrier()

    @pl.when(s < 15)
    def _():
        pltpu.sync_copy(acc_sh.at[pl.ds(s * CB, CB)], out_hbm.at[c, pl.ds(s * CB, CB)])

    @pl.when(s == 15)
    def _():
        pltpu.sync_copy(acc_sh.at[pl.ds(15 * CB, CBL)], out_hbm.at[c, pl.ds(15 * CB, CBL)])


_BM = 1000
_NB = N // _BM


def _tc1_body(x_ref, w_ref, deg_ref, hp_ref):
    dv = lax.rsqrt(deg_ref[0])
    h = jnp.dot(x_ref[0], w_ref[...], preferred_element_type=jnp.float32)
    hp_ref[0] = h * dv


def _tc1(x, w1, deg):
    return pl.pallas_call(
        _tc1_body,
        grid=(2, _NB),
        in_specs=[
            pl.BlockSpec((1, _BM, D), lambda g, i: (g, i, 0)),
            pl.BlockSpec((D, D), lambda g, i: (0, 0)),
            pl.BlockSpec((1, _BM, D), lambda g, i: (g, i, 0)),
        ],
        out_specs=pl.BlockSpec((1, _BM, D), lambda g, i: (g, i, 0)),
        out_shape=jax.ShapeDtypeStruct((2, N, D), jnp.float32),
    )(x, w1, deg)


def _tc2_body(agg_ref, deg_ref, b_ref, w_ref, out_ref):
    dv = lax.rsqrt(deg_ref[0])
    y = jnp.maximum(agg_ref[0] * dv + b_ref[...], 0.0)
    out_ref[0] = jnp.dot(y, w_ref[...], preferred_element_type=jnp.float32) * dv


def _tc2(agg, deg, b1, w2):
    return pl.pallas_call(
        _tc2_body,
        grid=(2, _NB),
        in_specs=[
            pl.BlockSpec((1, _BM, D), lambda g, i: (g, i, 0)),
            pl.BlockSpec((1, _BM, D), lambda g, i: (g, i, 0)),
            pl.BlockSpec((1, D), lambda g, i: (0, 0)),
            pl.BlockSpec((D, D), lambda g, i: (0, 0)),
        ],
        out_specs=pl.BlockSpec((1, _BM, D), lambda g, i: (g, i, 0)),
        out_shape=jax.ShapeDtypeStruct((2, N, D), jnp.float32),
    )(agg, deg, b1, w2)


def _tc3_body(agg_ref, deg_ref, b_ref, out_ref, sum_ref, sq_ref):
    p = pl.program_id(1)
    i = pl.program_id(2)
    dv = lax.rsqrt(deg_ref[0])
    h = agg_ref[0] * dv + b_ref[...]

    @pl.when((p == 0) & (i == 0))
    def _():
        sum_ref[...] = jnp.zeros_like(sum_ref)
        sq_ref[...] = jnp.zeros_like(sq_ref)

    @pl.when(p == 0)
    def _():
        sum_ref[...] += jnp.sum(h, axis=0, keepdims=True)
        sq_ref[...] += jnp.sum(h * h, axis=0, keepdims=True)
        out_ref[0] = h

    @pl.when(p == 1)
    def _():
        mean = sum_ref[...] / N
        var = (sq_ref[...] - N * mean * mean) / (N - 1)
        out_ref[0] = (h - mean) * lax.rsqrt(var)


def _tc3(agg, deg, b2):
    return pl.pallas_call(
        _tc3_body,
        grid=(2, 2, _NB),
        in_specs=[
            pl.BlockSpec((1, _BM, D), lambda g, p, i: (g, i, 0)),
            pl.BlockSpec((1, _BM, D), lambda g, p, i: (g, i, 0)),
            pl.BlockSpec((1, D), lambda g, p, i: (0, 0)),
        ],
        out_specs=pl.BlockSpec((1, _BM, D), lambda g, p, i: (g, i, 0)),
        out_shape=jax.ShapeDtypeStruct((2, N, D), jnp.float32),
        scratch_shapes=[
            pltpu.VMEM((1, D), jnp.float32),
            pltpu.VMEM((1, D), jnp.float32),
        ],
    )(agg, deg, b2)


def kernel(x1, x2, edge_index1, edge_index2, W1, b1, W2, b2):
    src = jnp.stack([edge_index1[0], edge_index2[0]])
    dst = jnp.stack([edge_index1[1], edge_index2[1]])
    extra = EPAD - E
    pad_src = (jnp.arange(extra, dtype=jnp.int32) * 97) % N
    pad_dst = N + (jnp.arange(extra, dtype=jnp.int32) % (NPAD - N))
    srcp = jnp.concatenate(
        [src, jnp.broadcast_to(pad_src, (2, extra))], axis=1
    ).reshape(2, NT, NCH, CH)
    dstp = jnp.concatenate(
        [dst, jnp.broadcast_to(pad_dst, (2, extra))], axis=1
    ).reshape(2, NT, NCH, CH)

    ones = jnp.ones((N, D), jnp.float32)
    x = jnp.stack([x1, x2])

    deg = _get_deg_kernel()(dstp, ones)
    hp1 = _tc1(x, W1, deg)
    agg = _get_agg_kernel()
    agg1 = agg(hp1, srcp, dstp)
    hp2 = _tc2(agg1, deg, b1.reshape(1, D), W2)
    agg2 = agg(hp2, srcp, dstp)
    z = _tc3(agg2, deg, b2.reshape(1, D))
    return z[0], z[1]
